# baseline (speedup 1.0000x reference)
import functools

import jax
import jax.numpy as jnp
from jax import lax
from jax.experimental import pallas as pl
from jax.experimental.pallas import tpu as pltpu
from jax.experimental.pallas import tpu_sc as plsc


def kernel(x, pe_weight):
    seq_len = x.shape[1]
    n_rows, dim = pe_weight.shape
    del n_rows

    info = plsc.get_sparse_core_info()
    nc, ns = info.num_cores, info.num_subcores
    nw = nc * ns
    rows_per_w = seq_len // nw
    rem = seq_len - rows_per_w * nw

    mesh = plsc.VectorSubcoreMesh(core_axis_name="c", subcore_axis_name="s")

    @functools.partial(
        pl.kernel,
        mesh=mesh,
        out_type=jax.ShapeDtypeStruct((seq_len, dim), pe_weight.dtype),
    )
    def copy_rows(table_hbm, out_hbm):
        wid = lax.axis_index("s") * nc + lax.axis_index("c")
        base = wid * rows_per_w
        pltpu.sync_copy(
            table_hbm.at[pl.ds(base, rows_per_w)],
            out_hbm.at[pl.ds(base, rows_per_w)],
        )
        if rem:
            @pl.when(wid == 0)
            def _():
                pltpu.sync_copy(
                    table_hbm.at[pl.ds(rows_per_w * nw, rem)],
                    out_hbm.at[pl.ds(rows_per_w * nw, rem)],
                )

    return copy_rows(pe_weight)[None, ...]


# speedup vs baseline: 1.0002x; 1.0002x over previous
import functools

import jax
import jax.numpy as jnp
from jax import lax
from jax.experimental import pallas as pl
from jax.experimental.pallas import tpu as pltpu
from jax.experimental.pallas import tpu_sc as plsc


def kernel(x, pe_weight):
    seq_len = x.shape[1]
    n_rows, dim = pe_weight.shape
    del n_rows

    info = plsc.get_sparse_core_info()
    nc, ns = info.num_cores, info.num_subcores
    nw = nc * ns
    rows_per_w = seq_len // nw
    rem = seq_len - rows_per_w * nw

    mesh = plsc.VectorSubcoreMesh(core_axis_name="c", subcore_axis_name="s")

    n_chunks = 8
    while rows_per_w % n_chunks:
        n_chunks //= 2
    rows_per_chunk = rows_per_w // n_chunks

    @functools.partial(
        pl.kernel,
        mesh=mesh,
        out_type=jax.ShapeDtypeStruct((seq_len, dim), pe_weight.dtype),
        scratch_types=[pltpu.SemaphoreType.DMA],
    )
    def copy_rows(table_hbm, out_hbm, sem):
        wid = lax.axis_index("s") * nc + lax.axis_index("c")
        base = wid * rows_per_w
        copies = []
        for j in range(n_chunks):
            off = base + j * rows_per_chunk
            copies.append(
                pltpu.async_copy(
                    table_hbm.at[pl.ds(off, rows_per_chunk)],
                    out_hbm.at[pl.ds(off, rows_per_chunk)],
                    sem,
                )
            )
        for c in copies:
            c.wait()
        if rem:
            @pl.when(wid == 0)
            def _():
                pltpu.sync_copy(
                    table_hbm.at[pl.ds(rows_per_w * nw, rem)],
                    out_hbm.at[pl.ds(rows_per_w * nw, rem)],
                )

    return copy_rows(pe_weight)[None, ...]


# speedup vs baseline: 1.0255x; 1.0253x over previous
import jax
import jax.numpy as jnp
from jax.experimental import pallas as pl
from jax.experimental.pallas import tpu as pltpu


def kernel(x, pe_weight):
    seq_len = x.shape[1]
    n_rows, dim = pe_weight.shape
    del n_rows

    def copy_body(src_hbm, out_hbm, sem):
        pltpu.make_async_copy(
            src_hbm.at[pl.ds(0, seq_len)], out_hbm, sem
        ).start()
        pltpu.make_async_copy(
            src_hbm.at[pl.ds(0, seq_len)], out_hbm, sem
        ).wait()

    out = pl.pallas_call(
        copy_body,
        out_shape=jax.ShapeDtypeStruct((seq_len, dim), pe_weight.dtype),
        in_specs=[pl.BlockSpec(memory_space=pltpu.MemorySpace.HBM)],
        out_specs=pl.BlockSpec(memory_space=pltpu.MemorySpace.HBM),
        scratch_shapes=[pltpu.SemaphoreType.DMA],
    )(pe_weight)
    return out[None, ...]


# speedup vs baseline: 42.1066x; 41.0606x over previous
import jax
import jax.numpy as jnp
from jax.experimental import pallas as pl
from jax.experimental.pallas import tpu as pltpu


def kernel(x, pe_weight):
    seq_len = x.shape[1]
    n_rows, dim = pe_weight.shape
    del n_rows

    block_rows = 512

    def copy_body(src_ref, out_ref):
        out_ref[...] = src_ref[...]

    out = pl.pallas_call(
        copy_body,
        grid=(seq_len // block_rows,),
        out_shape=jax.ShapeDtypeStruct((seq_len, dim), pe_weight.dtype),
        in_specs=[pl.BlockSpec((block_rows, dim), lambda i: (i, 0))],
        out_specs=pl.BlockSpec((block_rows, dim), lambda i: (i, 0)),
    )(pe_weight)
    return out[None, ...]


# speedup vs baseline: 45.8423x; 1.0887x over previous
import jax
import jax.numpy as jnp
from jax.experimental import pallas as pl
from jax.experimental.pallas import tpu as pltpu


def kernel(x, pe_weight):
    seq_len = x.shape[1]
    n_rows, dim = pe_weight.shape
    del n_rows

    block_rows = 1024

    def copy_body(src_ref, out_ref):
        out_ref[...] = src_ref[...]

    out = pl.pallas_call(
        copy_body,
        grid=(seq_len // block_rows,),
        out_shape=jax.ShapeDtypeStruct((seq_len, dim), pe_weight.dtype),
        in_specs=[pl.BlockSpec((block_rows, dim), lambda i: (i, 0))],
        out_specs=pl.BlockSpec((block_rows, dim), lambda i: (i, 0)),
    )(pe_weight)
    return out[None, ...]


# speedup vs baseline: 49.5640x; 1.0812x over previous
import jax
import jax.numpy as jnp
from jax.experimental import pallas as pl
from jax.experimental.pallas import tpu as pltpu


def kernel(x, pe_weight):
    seq_len = x.shape[1]
    n_rows, dim = pe_weight.shape
    del n_rows

    block_rows = 2048

    def copy_body(src_ref, out_ref):
        out_ref[...] = src_ref[...]

    out = pl.pallas_call(
        copy_body,
        grid=(seq_len // block_rows,),
        out_shape=jax.ShapeDtypeStruct((seq_len, dim), pe_weight.dtype),
        in_specs=[pl.BlockSpec((block_rows, dim), lambda i: (i, 0))],
        out_specs=pl.BlockSpec((block_rows, dim), lambda i: (i, 0)),
    )(pe_weight)
    return out[None, ...]
